# baseline (speedup 1.0000x reference)
import functools

import jax
import jax.numpy as jnp
from jax import lax
from jax.experimental import pallas as pl

C = 384
CR = 96
KNN = 8
REP = C // CR

CB = 32
GB = CB // REP


def _reduce_body(x_ref, wf_ref, out_ref):
    xb = x_ref[0]
    W = xb.shape[-1]
    s = jnp.sum(xb, axis=-1) * (1.0 / W)
    wf = wf_ref[0]
    weighted = s * wf
    g_iota = lax.broadcasted_iota(jnp.int32, (GB, CB), 0)
    c_iota = lax.broadcasted_iota(jnp.int32, (GB, CB), 1)
    P = jnp.where(c_iota // REP == g_iota, 1.0, 0.0)
    out_ref[0] = jnp.dot(P, weighted, preferred_element_type=jnp.float32)


def _graph_body(xr_ref, uk_ref, wgcn_ref, bcol_ref, out_ref):
    H = uk_ref.shape[1]
    U = uk_ref[0]
    ss = jnp.sum(U * U, axis=-1, keepdims=True)
    inv = 1.0 / jnp.maximum(jnp.sqrt(ss), 1e-12)
    Un = U * inv
    sim = lax.dot_general(Un, Un, (((1,), (1,)), ((), ())),
                          preferred_element_type=jnp.float32)

    col = lax.broadcasted_iota(jnp.int32, (H, H), 1)
    work = sim
    Aacc = jnp.zeros((H, H), jnp.float32)
    deg = jnp.zeros((H, 1), jnp.float32)
    for _ in range(KNN):
        m = jnp.max(work, axis=1, keepdims=True)
        cand = jnp.where(work == m, col, jnp.int32(1 << 30))
        jstar = jnp.min(cand, axis=1, keepdims=True)
        onehot = col == jstar
        Aacc = Aacc + jnp.where(onehot, work, 0.0)
        deg = deg + m
        work = jnp.where(onehot, jnp.float32(-1e30), work)

    dis = lax.rsqrt(deg)
    Xr = xr_ref[0]
    support = lax.dot_general(Xr, wgcn_ref[...], (((0,), (0,)), ((), ())),
                              preferred_element_type=jnp.float32)
    scaled = dis * support
    Afull = dis * Aacc
    msgT = lax.dot_general(scaled, Afull, (((0,), (1,)), ((), ())),
                           preferred_element_type=jnp.float32)
    out_ref[0] = jnp.maximum(msgT + bcol_ref[...], 0.0)


def _expand_body(on_ref, we_ref, out_ref):
    on = on_ref[0]
    H = on.shape[-1]
    Wdim = out_ref.shape[-1]
    we = we_ref[0]
    r_iota = lax.broadcasted_iota(jnp.int32, (CB, GB), 0)
    g_iota = lax.broadcasted_iota(jnp.int32, (CB, GB), 1)
    E = jnp.where(r_iota // REP == g_iota, 1.0, 0.0) * we
    scale = jnp.dot(E, on, preferred_element_type=jnp.float32)
    out_ref[0] = jnp.broadcast_to(scale[:, :, None], (CB, H, Wdim))


@jax.jit
def kernel(x, Uk, W_reduce, W_gcn, b_gcn, W_expand):
    B, Cc, H, W = x.shape
    NCB = Cc // CB

    w_flat3 = W_reduce.reshape(NCB, CB, 1)
    b_col = b_gcn.reshape(CR, 1)
    wexp3 = W_expand.reshape(NCB, CB, 1)

    Xr = pl.pallas_call(
        _reduce_body,
        grid=(B, NCB),
        in_specs=[
            pl.BlockSpec((1, CB, H, W), lambda b, c: (b, c, 0, 0)),
            pl.BlockSpec((1, CB, 1), lambda b, c: (c, 0, 0)),
        ],
        out_specs=pl.BlockSpec((1, GB, H), lambda b, c: (b, c, 0)),
        out_shape=jax.ShapeDtypeStruct((B, CR, H), jnp.float32),
    )(x, w_flat3)

    outT = pl.pallas_call(
        _graph_body,
        grid=(B,),
        in_specs=[
            pl.BlockSpec((1, CR, H), lambda b: (b, 0, 0)),
            pl.BlockSpec((1, H, Uk.shape[-1]), lambda b: (b, 0, 0)),
            pl.BlockSpec((CR, CR), lambda b: (0, 0)),
            pl.BlockSpec((CR, 1), lambda b: (0, 0)),
        ],
        out_specs=pl.BlockSpec((1, CR, H), lambda b: (b, 0, 0)),
        out_shape=jax.ShapeDtypeStruct((B, CR, H), jnp.float32),
    )(Xr, Uk, W_gcn, b_col)

    out = pl.pallas_call(
        _expand_body,
        grid=(B, NCB),
        in_specs=[
            pl.BlockSpec((1, GB, H), lambda b, c: (b, c, 0)),
            pl.BlockSpec((1, CB, 1), lambda b, c: (c, 0, 0)),
        ],
        out_specs=pl.BlockSpec((1, CB, H, W), lambda b, c: (b, c, 0, 0)),
        out_shape=jax.ShapeDtypeStruct((B, Cc, H, W), jnp.float32),
    )(outT, wexp3)
    return out


# speedup vs baseline: 4.4029x; 4.4029x over previous
import jax
import jax.numpy as jnp
from jax import lax
from jax.experimental import pallas as pl

C = 384
CR = 96
KNN = 8
REP = C // CR
HB = 32


def _reduce_body(x_ref, wf_ref, out_ref):
    xb = x_ref[0]
    W = xb.shape[1]
    s = jnp.sum(xb, axis=1) * (1.0 / W)
    sw = s * wf_ref[...]
    c_iota = lax.broadcasted_iota(jnp.int32, (C, CR), 0)
    g_iota = lax.broadcasted_iota(jnp.int32, (C, CR), 1)
    P = jnp.where(c_iota // REP == g_iota, 1.0, 0.0)
    out_ref[0] = jnp.dot(sw, P, preferred_element_type=jnp.float32)


def _graph_body(xr_ref, ukt_ref, wgcn_ref, brow_ref, out_ref):
    U = ukt_ref[0]
    H = U.shape[1]
    ss = jnp.sum(U * U, axis=0, keepdims=True)
    inv = 1.0 / jnp.maximum(jnp.sqrt(ss), 1e-12)
    Un = U * inv
    sim = lax.dot_general(Un, Un, (((0,), (0,)), ((), ())),
                          preferred_element_type=jnp.float32)

    col = lax.broadcasted_iota(jnp.int32, (H, H), 1)
    work = sim
    Aacc = jnp.zeros((H, H), jnp.float32)
    deg = jnp.zeros((H, 1), jnp.float32)
    for _ in range(KNN):
        m = jnp.max(work, axis=1, keepdims=True)
        cand = jnp.where(work == m, col, jnp.int32(1 << 30))
        jstar = jnp.min(cand, axis=1, keepdims=True)
        onehot = col == jstar
        Aacc = Aacc + jnp.where(onehot, work, 0.0)
        deg = deg + m
        work = jnp.where(onehot, jnp.float32(-1e30), work)

    dis = lax.rsqrt(deg)
    Xi = xr_ref[0]
    support = jnp.dot(Xi, wgcn_ref[...], preferred_element_type=jnp.float32)
    scaled = dis * support
    Afull = dis * Aacc
    msg = lax.dot_general(Afull, scaled, (((1,), (0,)), ((), ())),
                          preferred_element_type=jnp.float32)
    out_ref[0] = jnp.maximum(msg + brow_ref[...], 0.0)


def _expand_body(on_ref, we_ref, out_ref):
    on = on_ref[0]
    Wdim = out_ref.shape[2]
    g_iota = lax.broadcasted_iota(jnp.int32, (CR, C), 0)
    c_iota = lax.broadcasted_iota(jnp.int32, (CR, C), 1)
    E = jnp.where(c_iota // REP == g_iota, 1.0, 0.0) * we_ref[...]
    scale = jnp.dot(on, E, preferred_element_type=jnp.float32)
    out_ref[0] = jnp.broadcast_to(scale[:, None, :], (on.shape[0], Wdim, C))


@jax.jit
def kernel(x, Uk, W_reduce, W_gcn, b_gcn, W_expand):
    B, Cc, H, W = x.shape
    NHB = H // HB

    xt = jnp.transpose(x, (0, 2, 3, 1))
    ukt = jnp.transpose(Uk, (0, 2, 1))
    wf = W_reduce.reshape(1, Cc)
    we = W_expand.reshape(1, Cc)
    br = b_gcn.reshape(1, CR)

    Xr = pl.pallas_call(
        _reduce_body,
        grid=(B, NHB),
        in_specs=[
            pl.BlockSpec((1, HB, W, Cc), lambda b, h: (b, h, 0, 0)),
            pl.BlockSpec((1, Cc), lambda b, h: (0, 0)),
        ],
        out_specs=pl.BlockSpec((1, HB, CR), lambda b, h: (b, h, 0)),
        out_shape=jax.ShapeDtypeStruct((B, H, CR), jnp.float32),
    )(xt, wf)

    onodes = pl.pallas_call(
        _graph_body,
        grid=(B,),
        in_specs=[
            pl.BlockSpec((1, H, CR), lambda b: (b, 0, 0)),
            pl.BlockSpec((1, Uk.shape[-1], H), lambda b: (b, 0, 0)),
            pl.BlockSpec((CR, CR), lambda b: (0, 0)),
            pl.BlockSpec((1, CR), lambda b: (0, 0)),
        ],
        out_specs=pl.BlockSpec((1, H, CR), lambda b: (b, 0, 0)),
        out_shape=jax.ShapeDtypeStruct((B, H, CR), jnp.float32),
    )(Xr, ukt, W_gcn, br)

    yt = pl.pallas_call(
        _expand_body,
        grid=(B, NHB),
        in_specs=[
            pl.BlockSpec((1, HB, CR), lambda b, h: (b, h, 0)),
            pl.BlockSpec((1, Cc), lambda b, h: (0, 0)),
        ],
        out_specs=pl.BlockSpec((1, HB, W, Cc), lambda b, h: (b, h, 0, 0)),
        out_shape=jax.ShapeDtypeStruct((B, H, W, Cc), jnp.float32),
    )(onodes, we)
    return jnp.transpose(yt, (0, 3, 1, 2))
